# Initial kernel scaffold; baseline (speedup 1.0000x reference)
import jax
import jax.numpy as jnp
from jax.experimental import pallas as pl


def kernel(node_feat, edge_index, dist, W_edge, W_node):
    raise NotImplementedError("write your pallas kernel here")


# speedup vs baseline: 3.3244x; 3.3244x over previous
import functools

import jax
import jax.numpy as jnp
from jax import lax
from jax.experimental import pallas as pl
from jax.experimental.pallas import tpu as pltpu
from jax.experimental.pallas import tpu_sc as plsc

NC = 2
NS = 16
L = 16
NEG_SLOPE = 0.01
DELTA = 0.5


def _mm_body(nf_ref, w_ref, out_ref):
    out_ref[0] = jnp.dot(nf_ref[...], w_ref[0],
                         preferred_element_type=jnp.float32)


def _project(node_feat, w4, rb):
    n, in_dim = node_feat.shape
    q, _, h = w4.shape
    grid = (q, n // rb)
    return pl.pallas_call(
        _mm_body,
        grid=grid,
        in_specs=[
            pl.BlockSpec((rb, in_dim), lambda i, r: (r, 0)),
            pl.BlockSpec((1, in_dim, h), lambda i, r: (i, 0, 0)),
        ],
        out_specs=pl.BlockSpec((1, rb, h), lambda i, r: (i, r, 0)),
        out_shape=jax.ShapeDtypeStruct((q, n, h), jnp.float32),
    )(node_feat, w4)


def _out_body(nf_ref, ag_ref, wa_ref, wb_ref, out_ref):
    acc = jnp.dot(nf_ref[...], wa_ref[...],
                  preferred_element_type=jnp.float32)
    acc += jnp.dot(ag_ref[0], wb_ref[0], preferred_element_type=jnp.float32)
    acc += jnp.dot(ag_ref[1], wb_ref[1], preferred_element_type=jnp.float32)
    out_ref[...] = jnp.maximum(acc, NEG_SLOPE * acc)


def _node_update(node_feat, aggr2, wa, wb2, rb):
    n, in_dim = node_feat.shape
    h = aggr2.shape[2]
    out_dim = wa.shape[1]
    return pl.pallas_call(
        _out_body,
        grid=(n // rb,),
        in_specs=[
            pl.BlockSpec((rb, in_dim), lambda r: (r, 0)),
            pl.BlockSpec((2, rb, h), lambda r: (0, r, 0)),
            pl.BlockSpec((in_dim, out_dim), lambda r: (0, 0)),
            pl.BlockSpec((2, h, out_dim), lambda r: (0, 0, 0)),
        ],
        out_specs=pl.BlockSpec((rb, out_dim), lambda r: (r, 0)),
        out_shape=jax.ShapeDtypeStruct((n, out_dim), jnp.float32),
    )(node_feat, aggr2, wa, wb2)


def _make_edge_kernel(n, e, h, c_chunk):
    ept = e // NS
    nchunk = ept // c_chunk
    npt = (n // NS) // 8 * 8
    npt_last = n - (NS - 1) * npt
    ngrp = h // L
    mesh = plsc.VectorSubcoreMesh(core_axis_name="c", subcore_axis_name="s",
                                  num_cores=NC, num_subcores=NS)

    @functools.partial(
        pl.kernel,
        out_type=jax.ShapeDtypeStruct((NC, n, h), jnp.float32),
        mesh=mesh,
        scratch_types=[
            pltpu.VMEM((c_chunk,), jnp.int32),
            pltpu.VMEM((c_chunk,), jnp.int32),
            pltpu.VMEM((c_chunk,), jnp.float32),
            pltpu.VMEM((c_chunk, h), jnp.float32),
            pltpu.VMEM((c_chunk, h), jnp.float32),
            pltpu.VMEM((h,), jnp.float32),
            pltpu.VMEM_SHARED((n, h), jnp.float32),
            pltpu.SemaphoreType.DMA,
            pltpu.SemaphoreType.DMA,
        ],
    )
    def edge_kernel(t4, src_h, dst_h, dist_h, wd2, zeros, out,
                    src_v, dst_v, dist_v, pu_v, pv_v, wd_v, acc, sem0, sem1):
        c = lax.axis_index("c")
        s = lax.axis_index("s")

        @pl.when(s < NS - 1)
        def _():
            pltpu.sync_copy(zeros.at[pl.ds(0, npt)],
                            acc.at[pl.ds(s * npt, npt)])

        @pl.when(s == NS - 1)
        def _():
            pltpu.sync_copy(zeros, acc.at[pl.ds((NS - 1) * npt, npt_last)])

        pltpu.sync_copy(wd2.at[c], wd_v)
        plsc.subcore_barrier()
        wd_regs = [wd_v[pl.ds(L * j, L)] for j in range(ngrp)]

        def chunk_body(k, carry):
            base = s * ept + k * c_chunk
            pltpu.sync_copy(src_h.at[pl.ds(base, c_chunk)], src_v)
            pltpu.sync_copy(dst_h.at[pl.ds(base, c_chunk)], dst_v)
            pltpu.sync_copy(dist_h.at[pl.ds(base, c_chunk)], dist_v)
            cp_u = pltpu.async_copy(t4.at[c].at[src_v], pu_v, sem0)
            cp_v = pltpu.async_copy(t4.at[2 + c].at[dst_v], pv_v, sem1)
            cp_u.wait()
            cp_v.wait()

            def group_body(g, inner):
                dist16 = dist_v[pl.ds(g * L, L)]
                fac16 = jnp.where(dist16 < DELTA, 1.0, 0.0)
                for lane in range(L):
                    db = dist16[lane]
                    factor = fac16[lane]
                    ei = g * L + lane
                    for j in range(ngrp):
                        x = (pu_v[ei, pl.ds(L * j, L)]
                             + pv_v[ei, pl.ds(L * j, L)]
                             + db * wd_regs[j])
                        y = jnp.maximum(x, NEG_SLOPE * x)
                        pu_v[ei, pl.ds(L * j, L)] = y * factor
                return inner

            lax.fori_loop(0, c_chunk // L, group_body, 0)
            pltpu.sync_copy(pu_v, acc.at[dst_v], add=True)
            return carry

        lax.fori_loop(0, nchunk, chunk_body, 0)
        plsc.subcore_barrier()

        @pl.when(s < NS - 1)
        def _():
            pltpu.sync_copy(acc.at[pl.ds(s * npt, npt)],
                            out.at[c, pl.ds(s * npt, npt)])

        @pl.when(s == NS - 1)
        def _():
            pltpu.sync_copy(acc.at[pl.ds((NS - 1) * npt, npt_last)],
                            out.at[c, pl.ds((NS - 1) * npt, npt_last)])

    return edge_kernel


def kernel(node_feat, edge_index, dist, W_edge, W_node):
    n, in_dim = node_feat.shape
    e = dist.shape[0]
    out_dim = W_edge.shape[1]
    h = out_dim // 2

    wu = W_edge[:in_dim]
    wd = W_edge[in_dim]
    wv = W_edge[in_dim + 1:]
    w4 = jnp.stack([wu[:, :h], wu[:, h:], wv[:, :h], wv[:, h:]])
    wd2 = wd.reshape(2, h)

    t4 = _project(node_feat, w4, rb=2000)

    src = edge_index[0].astype(jnp.int32)
    dst = edge_index[1].astype(jnp.int32)
    npt_last = n - (NS - 1) * ((n // NS) // 8 * 8)
    zeros = jnp.zeros((npt_last, h), jnp.float32)
    edge_kernel = _make_edge_kernel(n, e, h, c_chunk=80)
    aggr2 = edge_kernel(t4, src, dst, dist, wd2, zeros)

    wa = W_node[:in_dim]
    wb2 = W_node[in_dim:].reshape(2, h, out_dim)
    return _node_update(node_feat, aggr2, wa, wb2, rb=2000)
